# baseline (speedup 1.0000x reference)
import functools

import jax
import jax.numpy as jnp
from jax import lax
from jax.experimental import pallas as pl
from jax.experimental.pallas import tpu as pltpu
from jax.experimental.pallas import tpu_sc as plsc

BATCH = 16384
LIN_IN = 13
N_CATS = 26
EMB_DIM = 16
HIDDEN = 256
OUT = 6

CATS_PAD = 32
N_GROUPS = CATS_PAD // 8
NUM_IDX = BATCH * CATS_PAD
NUM_WORKERS = 32
ROWS_PER_W = NUM_IDX // NUM_WORKERS
N_CHUNKS = 8
CHUNK = ROWS_PER_W // N_CHUNKS


def _gather_body(table_hbm, idx_hbm, out_hbm, idx_v, rows_v0, rows_v1,
                 gsem0, gsem1, osem0, osem1):
    wid = lax.axis_index("s") * 2 + lax.axis_index("c")
    base = wid * ROWS_PER_W
    pltpu.sync_copy(idx_hbm.at[pl.ds(base, ROWS_PER_W)], idx_v)
    rows = (rows_v0, rows_v1)
    gsems = (gsem0, gsem1)
    osems = (osem0, osem1)

    def start_gather(ci):
        b = ci % 2
        return pltpu.async_copy(
            table_hbm.at[idx_v.at[pl.ds(ci * CHUNK, CHUNK)]], rows[b], gsems[b]
        )

    def start_out(ci):
        b = ci % 2
        return pltpu.async_copy(
            rows[b], out_hbm.at[pl.ds(base + ci * CHUNK, CHUNK)], osems[b]
        )

    start_gather(0)
    for ci in range(N_CHUNKS):
        b = ci % 2
        pltpu.make_async_copy(
            table_hbm.at[idx_v.at[pl.ds(ci * CHUNK, CHUNK)]], rows[b], gsems[b]
        ).wait()
        if ci + 1 < N_CHUNKS:
            if ci >= 1:
                pltpu.make_async_copy(
                    rows[1 - b],
                    out_hbm.at[pl.ds(base + (ci - 1) * CHUNK, CHUNK)],
                    osems[1 - b],
                ).wait()
            start_gather(ci + 1)
        start_out(ci)
    pltpu.make_async_copy(
        rows[(N_CHUNKS - 1) % 2],
        out_hbm.at[pl.ds(base + (N_CHUNKS - 1) * CHUNK, CHUNK)],
        osems[(N_CHUNKS - 1) % 2],
    ).wait()
    pltpu.make_async_copy(
        rows[(N_CHUNKS - 2) % 2],
        out_hbm.at[pl.ds(base + (N_CHUNKS - 2) * CHUNK, CHUNK)],
        osems[(N_CHUNKS - 2) % 2],
    ).wait()


@functools.cache
def _make_gather():
    return pl.kernel(
        _gather_body,
        out_type=jax.ShapeDtypeStruct((NUM_IDX, EMB_DIM), jnp.float32),
        scratch_types=[
            pltpu.VMEM((ROWS_PER_W,), jnp.int32),
            pltpu.VMEM((CHUNK, EMB_DIM), jnp.float32),
            pltpu.VMEM((CHUNK, EMB_DIM), jnp.float32),
            pltpu.SemaphoreType.DMA,
            pltpu.SemaphoreType.DMA,
            pltpu.SemaphoreType.DMA,
            pltpu.SemaphoreType.DMA,
        ],
        mesh=plsc.VectorSubcoreMesh(core_axis_name="c", subcore_axis_name="s"),
        compiler_params=pltpu.CompilerParams(use_tc_tiling_on_sc=False),
    )


BM = 1024


def _mlp_body(x1_ref, e_ref, w1a_ref, w1b_ref, b1_ref, w2_ref, b2_ref, o_ref):
    acc = jnp.dot(x1_ref[...], w1a_ref[...], preferred_element_type=jnp.float32)
    for g in range(N_GROUPS):
        acc += jnp.dot(e_ref[g], w1b_ref[g], preferred_element_type=jnp.float32)
    h = jnp.tanh(acc + b1_ref[...])
    o_ref[...] = (
        jnp.dot(h, w2_ref[...], preferred_element_type=jnp.float32) + b2_ref[...]
    )


def _mlp(x1, e4, w1a, w1b4, b1, w2, b2):
    grid = (BATCH // BM,)
    return pl.pallas_call(
        _mlp_body,
        grid=grid,
        in_specs=[
            pl.BlockSpec((BM, LIN_IN), lambda i: (i, 0)),
            pl.BlockSpec((N_GROUPS, BM, 128), lambda i: (0, i, 0)),
            pl.BlockSpec((LIN_IN, HIDDEN), lambda i: (0, 0)),
            pl.BlockSpec((N_GROUPS, 128, HIDDEN), lambda i: (0, 0, 0)),
            pl.BlockSpec((1, HIDDEN), lambda i: (0, 0)),
            pl.BlockSpec((HIDDEN, OUT), lambda i: (0, 0)),
            pl.BlockSpec((1, OUT), lambda i: (0, 0)),
        ],
        out_specs=pl.BlockSpec((BM, OUT), lambda i: (i, 0)),
        out_shape=jax.ShapeDtypeStruct((BATCH, OUT), jnp.float32),
    )(x1, e4, w1a, w1b4, b1, w2, b2)


def kernel(x1, x2, emb, W1, b1, W2, b2):
    idx = jnp.pad(x2.astype(jnp.int32), ((0, 0), (0, CATS_PAD - N_CATS)))
    idx = idx.reshape(BATCH, N_GROUPS, 8).transpose(1, 0, 2).reshape(-1)
    e = _make_gather()(emb, idx)
    e4 = e.reshape(N_GROUPS, BATCH, 128)
    w1b4 = jnp.pad(W1[LIN_IN:], ((0, 128 * N_GROUPS - N_CATS * EMB_DIM), (0, 0)))
    w1b4 = w1b4.reshape(N_GROUPS, 128, HIDDEN)
    return _mlp(
        x1,
        e4,
        W1[:LIN_IN],
        w1b4,
        b1.reshape(1, HIDDEN),
        W2,
        b2.reshape(1, OUT),
    )


# speedup vs baseline: 1.5123x; 1.5123x over previous
import functools

import jax
import jax.numpy as jnp
from jax import lax
from jax.experimental import pallas as pl
from jax.experimental.pallas import tpu as pltpu
from jax.experimental.pallas import tpu_sc as plsc

BATCH = 16384
LIN_IN = 13
N_CATS = 26
EMB_DIM = 16
HIDDEN = 256
OUT = 6

CATS_PAD = 32
N_GROUPS = CATS_PAD // 8
NUM_IDX = BATCH * CATS_PAD
NUM_WORKERS = 32
ROWS_PER_W = NUM_IDX // NUM_WORKERS
N_CHUNKS = 4
CHUNK = ROWS_PER_W // N_CHUNKS


def _gather_body(table_hbm, idx_hbm, out_hbm, idx_v, rows_v, sem):
    wid = lax.axis_index("s") * 2 + lax.axis_index("c")
    base = wid * ROWS_PER_W
    pltpu.sync_copy(idx_hbm.at[pl.ds(base, ROWS_PER_W)], idx_v)
    for ci in range(N_CHUNKS):
        off = ci * CHUNK
        pltpu.async_copy(
            table_hbm.at[idx_v.at[pl.ds(off, CHUNK)]], rows_v, sem
        ).wait()
        pltpu.sync_copy(rows_v, out_hbm.at[pl.ds(base + off, CHUNK)])


@functools.cache
def _make_gather():
    return pl.kernel(
        _gather_body,
        out_type=jax.ShapeDtypeStruct((NUM_IDX, EMB_DIM), jnp.float32),
        scratch_types=[
            pltpu.VMEM((ROWS_PER_W,), jnp.int32),
            pltpu.VMEM((CHUNK, EMB_DIM), jnp.float32),
            pltpu.SemaphoreType.DMA,
        ],
        mesh=plsc.VectorSubcoreMesh(core_axis_name="c", subcore_axis_name="s"),
        compiler_params=pltpu.CompilerParams(use_tc_tiling_on_sc=False),
    )


BM = 1024


def _mlp_body(x1_ref, e_ref, w1a_ref, w1b_ref, b1_ref, w2_ref, b2_ref, o_ref):
    acc = jnp.dot(x1_ref[...], w1a_ref[...], preferred_element_type=jnp.float32)
    for g in range(N_GROUPS):
        acc += jnp.dot(e_ref[g], w1b_ref[g], preferred_element_type=jnp.float32)
    h = jnp.tanh(acc + b1_ref[...])
    o_ref[...] = (
        jnp.dot(h, w2_ref[...], preferred_element_type=jnp.float32) + b2_ref[...]
    )


def _mlp(x1, e4, w1a, w1b4, b1, w2, b2):
    grid = (BATCH // BM,)
    return pl.pallas_call(
        _mlp_body,
        grid=grid,
        in_specs=[
            pl.BlockSpec((BM, LIN_IN), lambda i: (i, 0)),
            pl.BlockSpec((N_GROUPS, BM, 128), lambda i: (0, i, 0)),
            pl.BlockSpec((LIN_IN, HIDDEN), lambda i: (0, 0)),
            pl.BlockSpec((N_GROUPS, 128, HIDDEN), lambda i: (0, 0, 0)),
            pl.BlockSpec((1, HIDDEN), lambda i: (0, 0)),
            pl.BlockSpec((HIDDEN, OUT), lambda i: (0, 0)),
            pl.BlockSpec((1, OUT), lambda i: (0, 0)),
        ],
        out_specs=pl.BlockSpec((BM, OUT), lambda i: (i, 0)),
        out_shape=jax.ShapeDtypeStruct((BATCH, OUT), jnp.float32),
    )(x1, e4, w1a, w1b4, b1, w2, b2)


def kernel(x1, x2, emb, W1, b1, W2, b2):
    idx = jnp.pad(x2.astype(jnp.int32), ((0, 0), (0, CATS_PAD - N_CATS)))
    idx = idx.reshape(BATCH, N_GROUPS, 8).transpose(1, 0, 2).reshape(-1)
    e = _make_gather()(emb, idx)
    e4 = e.reshape(N_GROUPS, BATCH, 128)
    w1b4 = jnp.pad(W1[LIN_IN:], ((0, 128 * N_GROUPS - N_CATS * EMB_DIM), (0, 0)))
    w1b4 = w1b4.reshape(N_GROUPS, 128, HIDDEN)
    return _mlp(
        x1,
        e4,
        W1[:LIN_IN],
        w1b4,
        b1.reshape(1, HIDDEN),
        W2,
        b2.reshape(1, OUT),
    )
